# baseline (speedup 1.0000x reference)
import functools

import jax
import jax.numpy as jnp
import numpy as np
from jax import lax
from jax.experimental import pallas as pl
from jax.experimental.pallas import tpu as pltpu
from jax.experimental.pallas import tpu_sc as plsc

B, S, D = 1, 2048, 768
E, K, NS = 8, 2, 2
H = 4 * D
A = S * K
TB = 256
NBR = A // TB + E
RPAD = NBR * TB
NSC = 2
NT = 16
NW = NSC * NT


def _gelu(h):
    return 0.5 * h * (1.0 + jnp.tanh(0.7978845608028654 * (h + 0.044715 * h * h * h)))


def _router_body(x_ref, c_ref, b_ref, lt_ref, tri_ref, pp_ref, wp_ref,
                 blk_ref):
    x = x_ref[...]
    c = c_ref[...]
    raw = lax.dot_general(x, c, (((1,), (1,)), ((), ())),
                          preferred_element_type=jnp.float32,
                          precision=lax.Precision.DEFAULT)
    sbal = jax.nn.sigmoid(raw + b_ref[...])
    lane8 = lax.broadcasted_iota(jnp.int32, (S, E), 1)
    m1 = jnp.max(sbal, axis=1, keepdims=True)
    i0 = jnp.min(jnp.where(sbal == m1, lane8, E), axis=1, keepdims=True)
    neg = jnp.where(lane8 == i0, -1.0, sbal)
    m2 = jnp.max(neg, axis=1, keepdims=True)
    i1 = jnp.min(jnp.where(neg == m2, lane8, E), axis=1, keepdims=True)
    sg = jax.nn.sigmoid(raw)
    g0 = jnp.sum(jnp.where(lane8 == i0, sg, 0.0), axis=1, keepdims=True)
    g1 = jnp.sum(jnp.where(lane8 == i1, sg, 0.0), axis=1, keepdims=True)
    p0 = jax.nn.sigmoid(g0 - g1)
    p1 = 1.0 - p0

    oh = ((lane8 == i0) | (lane8 == i1)).astype(jnp.float32)
    tri = tri_ref[...]
    carry = jnp.zeros((1, E), jnp.float32)
    chunks = []
    for c_ in range(S // 128):
        ohc = oh[c_ * 128:(c_ + 1) * 128, :]
        local = lax.dot_general(tri, ohc, (((1,), (0,)), ((), ())),
                                preferred_element_type=jnp.float32)
        chunks.append(local + carry)
        carry = carry + local[127:128, :]
    csum = jnp.concatenate(chunks, axis=0)
    cexcl = csum - oh
    cnt = carry
    nb = (cnt.astype(jnp.int32) + (TB - 1)) // TB
    start_blk = lax.dot_general(nb.astype(jnp.float32), lt_ref[...],
                                (((1,), (0,)), ((), ())),
                                preferred_element_type=jnp.float32)
    start_row = start_blk * float(TB)
    pick = lambda arr, idx: jnp.sum(jnp.where(lane8 == idx, arr, 0.0), axis=1,
                                    keepdims=True)
    rank0 = pick(cexcl, i0)
    rank1 = pick(cexcl, i1)
    srow_b = jnp.broadcast_to(start_row, (S, E))
    pp0 = pick(srow_b, i0) + rank0
    pp1 = pick(srow_b, i1) + rank1
    pp_ref[...] = jnp.concatenate([pp0, pp1], axis=1).astype(jnp.int32)
    wp_ref[...] = jnp.concatenate([p0, p1], axis=1)

    biota = lax.broadcasted_iota(jnp.int32, (1, 128), 1)
    total_blk = jnp.sum(nb)
    acc = jnp.zeros((1, 128), jnp.int32)
    for e in range(E):
        acc = acc + (start_blk[0, e].astype(jnp.int32) <= biota).astype(jnp.int32)
    blk_e = acc - 1
    active = (biota < total_blk).astype(jnp.int32)
    blk_ref[...] = jnp.concatenate([blk_e, active], axis=0)


_SCH = A // NT // 2
_GR = RPAD // NW
_GCH = _GR // 3


def _dispatch_body(pp_hbm, tv_hbm, wp_hbm, x_hbm, xg_hbm, wg_hbm,
                   pp_v, tv_v, wv_v, idx_v, rows_a, rows_b, wout_v,
                   tok_sh, wg_sh, s0, s1, s2, sg0, sg1, sw0, sw1):
    cid = lax.axis_index("c")
    tid = lax.axis_index("s")
    base0 = tid * (2 * _SCH)
    base1 = base0 + _SCH
    cps = [
        pltpu.async_copy(pp_hbm.at[pl.ds(base0, _SCH)], pp_v.at[0], s0),
        pltpu.async_copy(pp_hbm.at[pl.ds(base1, _SCH)], pp_v.at[1], s0),
        pltpu.async_copy(tv_hbm.at[pl.ds(base0, _SCH)], tv_v.at[0], s1),
        pltpu.async_copy(tv_hbm.at[pl.ds(base1, _SCH)], tv_v.at[1], s1),
        pltpu.async_copy(wp_hbm.at[pl.ds(base0, _SCH)], wv_v.at[0], s2),
        pltpu.async_copy(wp_hbm.at[pl.ds(base1, _SCH)], wv_v.at[1], s2),
    ]
    for cp in cps:
        cp.wait()
    for r in range(2):
        pltpu.sync_copy(tv_v.at[r], tok_sh.at[pp_v.at[r]])
        pltpu.sync_copy(wv_v.at[r], wg_sh.at[pp_v.at[r]])
    plsc.subcore_barrier()
    wid = cid * NT + tid
    base = wid * _GR
    pltpu.sync_copy(tok_sh.at[pl.ds(base, _GR)], idx_v)
    for j in range(_GR // 16):
        v = idx_v[pl.ds(j * 16, 16)]
        idx_v[pl.ds(j * 16, 16)] = jnp.minimum(jnp.maximum(v, 0), S - 1)
    g0 = pltpu.async_copy(x_hbm.at[idx_v.at[pl.ds(0, _GCH)]], rows_a, sg0)
    g1 = pltpu.async_copy(x_hbm.at[idx_v.at[pl.ds(_GCH, _GCH)]], rows_b, sg1)
    g0.wait()
    w0 = pltpu.async_copy(rows_a, xg_hbm.at[pl.ds(base, _GCH)], sw0)
    g1.wait()
    w1 = pltpu.async_copy(rows_b, xg_hbm.at[pl.ds(base + _GCH, _GCH)], sw1)
    w0.wait()
    g2 = pltpu.async_copy(x_hbm.at[idx_v.at[pl.ds(2 * _GCH, _GCH)]], rows_a,
                          sg0)
    pltpu.sync_copy(wg_sh.at[pl.ds(base, _GR)], wout_v)
    g2.wait()
    w2 = pltpu.async_copy(rows_a, xg_hbm.at[pl.ds(base + 2 * _GCH, _GCH)], sw0)
    pltpu.sync_copy(wout_v, wg_hbm.at[pl.ds(base, _GR)])
    w1.wait()
    w2.wait()


TBS = 512


def _ffn_shared_body(x_ref, fc_ref, proj_ref, o_ref):
    s = pl.program_id(0)
    sb = pl.program_id(1)
    xb = x_ref[...]
    h = lax.dot_general(xb.astype(jnp.bfloat16), fc_ref[0],
                        (((1,), (1,)), ((), ())),
                        preferred_element_type=jnp.float32)
    h = _gelu(h)
    y = lax.dot_general(h.astype(jnp.bfloat16), proj_ref[0],
                        (((1,), (1,)), ((), ())),
                        preferred_element_type=jnp.float32)
    rows = pl.ds(sb * TBS, TBS)

    @pl.when(s == 0)
    def _():
        o_ref[rows, :] = xb + y

    @pl.when(s == 1)
    def _():
        o_ref[rows, :] += y


def _ffn_routed_body(blk_e_ref, act_ref, xg_ref, fc_ref, proj_ref, wg_ref,
                     o_ref):
    b = pl.program_id(0)

    @pl.when(act_ref[b] == 1)
    def _():
        xb = xg_ref[...]
        h = lax.dot_general(xb.astype(jnp.bfloat16), fc_ref[0],
                            (((1,), (1,)), ((), ())),
                            preferred_element_type=jnp.float32)
        h = _gelu(h)
        y = lax.dot_general(h.astype(jnp.bfloat16), proj_ref[0],
                            (((1,), (1,)), ((), ())),
                            preferred_element_type=jnp.float32)
        o_ref[...] = (y * wg_ref[...]).astype(jnp.bfloat16)

    @pl.when(act_ref[b] == 0)
    def _():
        o_ref[...] = jnp.zeros((TB, D), jnp.bfloat16)


def _combine_tc_body(shr_ref, pp_ref, ygr_ref, o_ref):
    ppb = pp_ref[...]
    liota = lax.broadcasted_iota(jnp.int32, (TB, RPAD), 1)
    q = ((liota == ppb[:, 0:1]) | (liota == ppb[:, 1:2])).astype(jnp.bfloat16)
    ysum = lax.dot_general(q, ygr_ref[...], (((1,), (0,)), ((), ())),
                           preferred_element_type=jnp.float32)
    o_ref[...] = shr_ref[...] + ysum


_CCH = 16


def _combine_body(shr_hbm, ygr_hbm, pp_hbm, out_hbm,
                  shv, ppv, gv, ov, ssh, spp, sg, swa, swb):
    wid = lax.axis_index("c") * NT + lax.axis_index("s")
    nr = S // NW // _CCH

    def fire(r):
        base = wid * (S // NW) + r * _CCH
        p = r % 2
        csh = pltpu.async_copy(shr_hbm.at[pl.ds(base, _CCH)], shv.at[p], ssh)
        cpp = pltpu.async_copy(pp_hbm.at[pl.ds(2 * base, 2 * _CCH)],
                               ppv.at[p], spp)
        cpp.wait()
        cg = pltpu.async_copy(ygr_hbm.at[ppv.at[p]], gv.at[p], sg)
        return csh, cg

    pend = fire(0)
    wsem = (swa, swb)
    wb = [None, None]
    for r in range(nr):
        p = r % 2
        csh, cg = pend
        if r + 1 < nr:
            pend = fire(r + 1)
        csh.wait()
        cg.wait()
        if wb[p] is not None:
            wb[p].wait()
        for i in range(_CCH):
            def body(j, _):
                sl = pl.ds(j * 16, 16)
                ov[p, i, sl] = shv[p, i, sl] + gv[p, 2 * i, sl] + gv[p, 2 * i + 1, sl]
                return 0
            lax.fori_loop(0, D // 16, body, 0)
        base = wid * (S // NW) + r * _CCH
        wb[p] = pltpu.async_copy(ov.at[p], out_hbm.at[pl.ds(base, _CCH)],
                                 wsem[p])
    wb[0].wait()
    wb[1].wait()


def kernel(x, shared_fc, shared_proj, routed_fc, routed_proj, centroids,
           routing_biases):
    x2 = x.reshape(S, D)
    fc_r = routed_fc.astype(jnp.bfloat16)
    proj_r = routed_proj.astype(jnp.bfloat16)
    fc_s = shared_fc.astype(jnp.bfloat16)
    proj_s = shared_proj.astype(jnp.bfloat16)
    bias2d = routing_biases.reshape(1, E)
    lt = jnp.asarray(np.triu(np.ones((E, E), np.float32), 1), jnp.float32)
    tri = jnp.asarray(np.tril(np.ones((128, 128), np.float32)), jnp.float32)
    tvals = jnp.asarray(np.arange(A, dtype=np.int32) // K)

    pp, wp, blk = pl.pallas_call(
        _router_body,
        out_shape=(
            jax.ShapeDtypeStruct((S, K), jnp.int32),
            jax.ShapeDtypeStruct((S, K), jnp.float32),
            jax.ShapeDtypeStruct((2, 128), jnp.int32),
        ),
    )(x2, centroids, bias2d, lt, tri)
    pp_flat = pp.reshape(A)
    wp_flat = wp.reshape(A)
    blk_e = blk[0, :NBR]
    blk_act = blk[1, :NBR]

    mesh = plsc.VectorSubcoreMesh(core_axis_name="c", subcore_axis_name="s")

    shr = pl.pallas_call(
        _ffn_shared_body,
        grid=(NS, S // TBS),
        in_specs=[
            pl.BlockSpec((TBS, D), lambda s, sb: (sb, 0)),
            pl.BlockSpec((1, H, D), lambda s, sb: (s, 0, 0)),
            pl.BlockSpec((1, D, H), lambda s, sb: (s, 0, 0)),
        ],
        out_specs=pl.BlockSpec((S, D), lambda s, sb: (0, 0)),
        out_shape=jax.ShapeDtypeStruct((S, D), jnp.float32),
    )(x2, fc_s, proj_s)

    xg, wg_pad = pl.kernel(
        _dispatch_body,
        mesh=mesh,
        out_type=(
            jax.ShapeDtypeStruct((RPAD, D), jnp.float32),
            jax.ShapeDtypeStruct((RPAD,), jnp.float32),
        ),
        scratch_types=[
            pltpu.VMEM((2, _SCH), jnp.int32),
            pltpu.VMEM((2, _SCH), jnp.int32),
            pltpu.VMEM((2, _SCH), jnp.float32),
            pltpu.VMEM((_GR,), jnp.int32),
            pltpu.VMEM((_GCH, D), jnp.float32),
            pltpu.VMEM((_GCH, D), jnp.float32),
            pltpu.VMEM((_GR,), jnp.float32),
            pltpu.VMEM_SHARED((RPAD,), jnp.int32),
            pltpu.VMEM_SHARED((RPAD,), jnp.float32),
            pltpu.SemaphoreType.DMA,
            pltpu.SemaphoreType.DMA,
            pltpu.SemaphoreType.DMA,
            pltpu.SemaphoreType.DMA,
            pltpu.SemaphoreType.DMA,
            pltpu.SemaphoreType.DMA,
            pltpu.SemaphoreType.DMA,
        ],
    )(pp_flat, tvals, wp_flat, x2)

    ygr = pl.pallas_call(
        _ffn_routed_body,
        grid_spec=pltpu.PrefetchScalarGridSpec(
            num_scalar_prefetch=2,
            grid=(NBR,),
            in_specs=[
                pl.BlockSpec((TB, D), lambda b, be, act: (b, 0)),
                pl.BlockSpec((1, H, D), lambda b, be, act: (be[b], 0, 0)),
                pl.BlockSpec((1, D, H), lambda b, be, act: (be[b], 0, 0)),
                pl.BlockSpec((TB, 1), lambda b, be, act: (b, 0)),
            ],
            out_specs=pl.BlockSpec((TB, D), lambda b, be, act: (b, 0)),
        ),
        out_shape=jax.ShapeDtypeStruct((RPAD, D), jnp.bfloat16),
    )(blk_e, blk_act, xg, fc_r, proj_r, wg_pad.reshape(RPAD, 1))

    out = pl.pallas_call(
        _combine_tc_body,
        grid=(S // TB,),
        in_specs=[
            pl.BlockSpec((TB, D), lambda sb: (sb, 0)),
            pl.BlockSpec((TB, K), lambda sb: (sb, 0)),
            pl.BlockSpec((RPAD, D), lambda sb: (0, 0)),
        ],
        out_specs=pl.BlockSpec((TB, D), lambda sb: (sb, 0)),
        out_shape=jax.ShapeDtypeStruct((S, D), jnp.float32),
    )(shr, pp, ygr)

    return out.reshape(B, S, D)


# speedup vs baseline: 1.1875x; 1.1875x over previous
import functools

import jax
import jax.numpy as jnp
import numpy as np
from jax import lax
from jax.experimental import pallas as pl
from jax.experimental.pallas import tpu as pltpu

B, S, D = 1, 2048, 768
E, K, NS = 8, 2, 2
H = 4 * D
A = S * K
TB = 256
NBR = A // TB + E
RPAD = NBR * TB


def _gelu(h):
    return 0.5 * h * (1.0 + jnp.tanh(0.7978845608028654 * (h + 0.044715 * h * h * h)))


def _router_body(x_ref, c_ref, b_ref, lt_ref, tri_ref, pp_ref, wp_ref,
                 blk_ref):
    x = x_ref[...]
    c = c_ref[...]
    raw = lax.dot_general(x, c, (((1,), (1,)), ((), ())),
                          preferred_element_type=jnp.float32,
                          precision=lax.Precision.DEFAULT)
    sbal = jax.nn.sigmoid(raw + b_ref[...])
    lane8 = lax.broadcasted_iota(jnp.int32, (S, E), 1)
    m1 = jnp.max(sbal, axis=1, keepdims=True)
    i0 = jnp.min(jnp.where(sbal == m1, lane8, E), axis=1, keepdims=True)
    neg = jnp.where(lane8 == i0, -1.0, sbal)
    m2 = jnp.max(neg, axis=1, keepdims=True)
    i1 = jnp.min(jnp.where(neg == m2, lane8, E), axis=1, keepdims=True)
    sg = jax.nn.sigmoid(raw)
    g0 = jnp.sum(jnp.where(lane8 == i0, sg, 0.0), axis=1, keepdims=True)
    g1 = jnp.sum(jnp.where(lane8 == i1, sg, 0.0), axis=1, keepdims=True)
    p0 = jax.nn.sigmoid(g0 - g1)
    p1 = 1.0 - p0

    oh = ((lane8 == i0) | (lane8 == i1)).astype(jnp.float32)
    tri = tri_ref[...]
    carry = jnp.zeros((1, E), jnp.float32)
    chunks = []
    for c_ in range(S // 128):
        ohc = oh[c_ * 128:(c_ + 1) * 128, :]
        local = lax.dot_general(tri, ohc, (((1,), (0,)), ((), ())),
                                preferred_element_type=jnp.float32)
        chunks.append(local + carry)
        carry = carry + local[127:128, :]
    csum = jnp.concatenate(chunks, axis=0)
    cexcl = csum - oh
    cnt = carry
    nb = (cnt.astype(jnp.int32) + (TB - 1)) // TB
    start_blk = lax.dot_general(nb.astype(jnp.float32), lt_ref[...],
                                (((1,), (0,)), ((), ())),
                                preferred_element_type=jnp.float32)
    start_row = start_blk * float(TB)
    pick = lambda arr, idx: jnp.sum(jnp.where(lane8 == idx, arr, 0.0), axis=1,
                                    keepdims=True)
    rank0 = pick(cexcl, i0)
    rank1 = pick(cexcl, i1)
    srow_b = jnp.broadcast_to(start_row, (S, E))
    pp0 = pick(srow_b, i0) + rank0
    pp1 = pick(srow_b, i1) + rank1
    pp_ref[...] = jnp.concatenate([pp0, pp1], axis=1).astype(jnp.int32)
    wp_ref[...] = jnp.concatenate([p0, p1], axis=1)

    biota = lax.broadcasted_iota(jnp.int32, (1, 128), 1)
    total_blk = jnp.sum(nb)
    acc = jnp.zeros((1, 128), jnp.int32)
    for e in range(E):
        acc = acc + (start_blk[0, e].astype(jnp.int32) <= biota).astype(jnp.int32)
    blk_e = acc - 1
    active = (biota < total_blk).astype(jnp.int32)
    blk_ref[...] = jnp.concatenate([blk_e, active], axis=0)


TBS = 512


def _ffn_shared_body(x_ref, fc_ref, proj_ref, o_ref):
    s = pl.program_id(0)
    sb = pl.program_id(1)
    xb = x_ref[...]
    h = lax.dot_general(xb.astype(jnp.bfloat16), fc_ref[0],
                        (((1,), (1,)), ((), ())),
                        preferred_element_type=jnp.float32)
    h = _gelu(h)
    y = lax.dot_general(h.astype(jnp.bfloat16), proj_ref[0],
                        (((1,), (1,)), ((), ())),
                        preferred_element_type=jnp.float32)
    rows = pl.ds(sb * TBS, TBS)

    @pl.when(s == 0)
    def _():
        o_ref[rows, :] = xb + y

    @pl.when(s == 1)
    def _():
        o_ref[rows, :] += y


def _ffn_routed_body(blk_e_ref, act_ref, ppt_ref, wpt_ref, xb_ref, fc_ref,
                     proj_ref, o_ref):
    b = pl.program_id(0)

    @pl.when(act_ref[b] == 1)
    def _():
        riota = b * TB + lax.broadcasted_iota(jnp.int32, (TB, S), 0)
        m0 = ppt_ref[0:1, :] == riota
        m1 = ppt_ref[1:2, :] == riota
        p = (m0 | m1).astype(jnp.bfloat16)
        xg = lax.dot_general(p, xb_ref[...], (((1,), (0,)), ((), ())),
                             preferred_element_type=jnp.float32)
        wcol = jnp.sum(jnp.where(m0, wpt_ref[0:1, :], 0.0)
                       + jnp.where(m1, wpt_ref[1:2, :], 0.0),
                       axis=1, keepdims=True)
        h = lax.dot_general(xg.astype(jnp.bfloat16), fc_ref[0],
                            (((1,), (1,)), ((), ())),
                            preferred_element_type=jnp.float32)
        h = _gelu(h)
        y = lax.dot_general(h.astype(jnp.bfloat16), proj_ref[0],
                            (((1,), (1,)), ((), ())),
                            preferred_element_type=jnp.float32)
        o_ref[...] = (y * wcol).astype(jnp.bfloat16)

    @pl.when(act_ref[b] == 0)
    def _():
        o_ref[...] = jnp.zeros((TB, D), jnp.bfloat16)


def _combine_tc_body(shr_ref, pp_ref, ygr_ref, o_ref):
    ppb = pp_ref[...]
    liota = lax.broadcasted_iota(jnp.int32, (TB, RPAD), 1)
    q = ((liota == ppb[:, 0:1]) | (liota == ppb[:, 1:2])).astype(jnp.bfloat16)
    ysum = lax.dot_general(q, ygr_ref[...], (((1,), (0,)), ((), ())),
                           preferred_element_type=jnp.float32)
    o_ref[...] = shr_ref[...] + ysum


def kernel(x, shared_fc, shared_proj, routed_fc, routed_proj, centroids,
           routing_biases):
    x2 = x.reshape(S, D)
    fc_r = routed_fc.astype(jnp.bfloat16)
    proj_r = routed_proj.astype(jnp.bfloat16)
    fc_s = shared_fc.astype(jnp.bfloat16)
    proj_s = shared_proj.astype(jnp.bfloat16)
    bias2d = routing_biases.reshape(1, E)
    lt = jnp.asarray(np.triu(np.ones((E, E), np.float32), 1), jnp.float32)
    tri = jnp.asarray(np.tril(np.ones((128, 128), np.float32)), jnp.float32)

    pp, wp, blk = pl.pallas_call(
        _router_body,
        out_shape=(
            jax.ShapeDtypeStruct((S, K), jnp.int32),
            jax.ShapeDtypeStruct((S, K), jnp.float32),
            jax.ShapeDtypeStruct((2, 128), jnp.int32),
        ),
    )(x2, centroids, bias2d, lt, tri)
    blk_e = blk[0, :NBR]
    blk_act = blk[1, :NBR]
    ppt = pp.T
    wpt = wp.T
    xb16 = x2.astype(jnp.bfloat16)

    shr = pl.pallas_call(
        _ffn_shared_body,
        grid=(NS, S // TBS),
        in_specs=[
            pl.BlockSpec((TBS, D), lambda s, sb: (sb, 0)),
            pl.BlockSpec((1, H, D), lambda s, sb: (s, 0, 0)),
            pl.BlockSpec((1, D, H), lambda s, sb: (s, 0, 0)),
        ],
        out_specs=pl.BlockSpec((S, D), lambda s, sb: (0, 0)),
        out_shape=jax.ShapeDtypeStruct((S, D), jnp.float32),
    )(x2, fc_s, proj_s)

    ygr = pl.pallas_call(
        _ffn_routed_body,
        grid_spec=pltpu.PrefetchScalarGridSpec(
            num_scalar_prefetch=2,
            grid=(NBR,),
            in_specs=[
                pl.BlockSpec((K, S), lambda b, be, act: (0, 0)),
                pl.BlockSpec((K, S), lambda b, be, act: (0, 0)),
                pl.BlockSpec((S, D), lambda b, be, act: (0, 0)),
                pl.BlockSpec((1, H, D), lambda b, be, act: (be[b], 0, 0)),
                pl.BlockSpec((1, D, H), lambda b, be, act: (be[b], 0, 0)),
            ],
            out_specs=pl.BlockSpec((TB, D), lambda b, be, act: (b, 0)),
        ),
        out_shape=jax.ShapeDtypeStruct((RPAD, D), jnp.bfloat16),
    )(blk_e, blk_act, ppt, wpt, xb16, fc_r, proj_r)

    out = pl.pallas_call(
        _combine_tc_body,
        grid=(S // TB,),
        in_specs=[
            pl.BlockSpec((TB, D), lambda sb: (sb, 0)),
            pl.BlockSpec((TB, K), lambda sb: (sb, 0)),
            pl.BlockSpec((RPAD, D), lambda sb: (0, 0)),
        ],
        out_specs=pl.BlockSpec((TB, D), lambda sb: (sb, 0)),
        out_shape=jax.ShapeDtypeStruct((S, D), jnp.float32),
    )(shr, pp, ygr)

    return out.reshape(B, S, D)
